# baseline (speedup 1.0000x reference)
import functools

import jax
import jax.numpy as jnp
from jax import lax
from jax.experimental import pallas as pl
from jax.experimental.pallas import tpu as pltpu
from jax.experimental.pallas import tpu_sc as plsc

BATCH = 4096
SEQ = 200
D = 64
NC = 2
NS = 16
NW = NC * NS
ROWS = BATCH * SEQ
RPW = ROWS // NW
CHUNKS = RPW // SEQ
G1 = 104
G2 = SEQ - G1
LANES = 16


def _sc_body(idx_h, table_h, pe_h, out_h, idx_v, pe_v, buf, sem):
    wid = lax.axis_index("s") * NC + lax.axis_index("c")
    base = wid * RPW

    pltpu.sync_copy(idx_h.at[pl.ds(base, RPW)], idx_v)
    pltpu.sync_copy(pe_h, pe_v)

    @pl.loop(0, CHUNKS)
    def _chunk(c):
        row0 = c * SEQ
        h1 = pltpu.async_copy(
            table_h.at[idx_v.at[pl.ds(row0, G1)]],
            buf.at[pl.ds(0, G1)], sem)
        h2 = pltpu.async_copy(
            table_h.at[idx_v.at[pl.ds(row0 + G1, G2)]],
            buf.at[pl.ds(G1, G2)], sem)
        h1.wait()
        h2.wait()

        @pl.loop(0, SEQ)
        def _row(r):
            for j in range(D // LANES):
                sl = pl.ds(j * LANES, LANES)
                buf[r, sl] = buf[r, sl] + pe_v[r, sl]

        pltpu.sync_copy(buf, out_h.at[pl.ds(base + row0, SEQ)])


@jax.jit
def _run(x_flat, table, pe_seq):
    mesh = plsc.VectorSubcoreMesh(
        core_axis_name="c", subcore_axis_name="s", num_cores=NC,
        num_subcores=NS)
    grid_kernel = pl.kernel(
        _sc_body,
        out_type=jax.ShapeDtypeStruct((ROWS, D), jnp.float32),
        mesh=mesh,
        scratch_types=[
            pltpu.VMEM((RPW,), jnp.int32),
            pltpu.VMEM((SEQ, D), jnp.float32),
            pltpu.VMEM((SEQ, D), jnp.float32),
            pltpu.SemaphoreType.DMA,
        ],
        compiler_params=pltpu.CompilerParams(use_tc_tiling_on_sc=False),
    )
    return grid_kernel(x_flat, table, pe_seq)


def kernel(x, table, pe):
    x_flat = x.reshape(ROWS)
    out = _run(x_flat, table, pe[:SEQ])
    return out.reshape(BATCH, SEQ, D)
